# baseline (speedup 1.0000x reference)
import jax
import jax.numpy as jnp
from jax.experimental import pallas as pl
from jax.experimental.pallas import tpu as pltpu

_BANDWIDTH = 2.0
_N_BUCKETS = 1024.0


def _bucketize(proj):
    f = jnp.floor(proj * (1.0 / _BANDWIDTH))
    return f - jnp.floor(f * (1.0 / _N_BUCKETS)) * _N_BUCKETS


def kernel(x, random_vectors):
    n, dim = x.shape
    n_hashes = random_vectors.shape[1]
    tile_m = 1024

    def outer(xa_hbm, xb_hbm, rv_vmem, o_hbm):
        def inner(xa_blk, xb_blk, o_blk):
            rv = rv_vmem[...]
            pa = jnp.dot(xa_blk[...], rv, preferred_element_type=jnp.float32)
            pb = jnp.dot(xb_blk[...], rv, preferred_element_type=jnp.float32)
            o_blk[:tile_m, :] = _bucketize(pa)
            o_blk[tile_m:, :] = _bucketize(pb)

        pltpu.emit_pipeline(
            inner,
            grid=(n // (2 * tile_m),),
            in_specs=[
                pl.BlockSpec((tile_m, dim), lambda i: (2 * i, 0),
                             pipeline_mode=pl.Buffered(buffer_count=8)),
                pl.BlockSpec((tile_m, dim), lambda i: (2 * i + 1, 0),
                             pipeline_mode=pl.Buffered(buffer_count=8)),
            ],
            out_specs=[
                pl.BlockSpec((2 * tile_m, n_hashes), lambda i: (i, 0),
                             pipeline_mode=pl.Buffered(buffer_count=2)),
            ],
        )(xa_hbm, xb_hbm, o_hbm)

    return pl.pallas_call(
        outer,
        in_specs=[
            pl.BlockSpec(memory_space=pltpu.HBM),
            pl.BlockSpec(memory_space=pltpu.HBM),
            pl.BlockSpec(memory_space=pltpu.VMEM),
        ],
        out_specs=pl.BlockSpec(memory_space=pltpu.HBM),
        out_shape=jax.ShapeDtypeStruct((n, n_hashes), jnp.float32),
    )(x, x, random_vectors)


# speedup vs baseline: 1.0043x; 1.0043x over previous
import jax
import jax.numpy as jnp
from jax.experimental import pallas as pl
from jax.experimental.pallas import tpu as pltpu

_BANDWIDTH = 2.0
_N_BUCKETS = 1024.0


def _bucketize(proj):
    f = jnp.floor(proj * (1.0 / _BANDWIDTH))
    return f - jnp.floor(f * (1.0 / _N_BUCKETS)) * _N_BUCKETS


def kernel(x, random_vectors):
    n, dim = x.shape
    n_hashes = random_vectors.shape[1]
    tile_m = 1024

    def outer(x_hbm, rv_vmem, o_hbm):
        def inner(x_blk, o_blk):
            proj = jnp.dot(x_blk[...], rv_vmem[...],
                           preferred_element_type=jnp.float32)
            o_blk[...] = _bucketize(proj)

        pltpu.emit_pipeline(
            inner,
            grid=(n // tile_m,),
            in_specs=[
                pl.BlockSpec((tile_m, dim), lambda i: (i, 0),
                             pipeline_mode=pl.Buffered(buffer_count=16)),
            ],
            out_specs=[
                pl.BlockSpec((tile_m, n_hashes), lambda i: (i, 0),
                             pipeline_mode=pl.Buffered(buffer_count=2)),
            ],
        )(x_hbm, o_hbm)

    return pl.pallas_call(
        outer,
        in_specs=[
            pl.BlockSpec(memory_space=pltpu.HBM),
            pl.BlockSpec(memory_space=pltpu.VMEM),
        ],
        out_specs=pl.BlockSpec(memory_space=pltpu.HBM),
        out_shape=jax.ShapeDtypeStruct((n, n_hashes), jnp.float32),
    )(x, random_vectors)
